# Initial kernel scaffold; baseline (speedup 1.0000x reference)
import jax
import jax.numpy as jnp
from jax.experimental import pallas as pl


def kernel(feat, edge_index, W_self, W_neigh, bias):
    raise NotImplementedError("write your pallas kernel here")


# speedup vs baseline: 5.8290x; 5.8290x over previous
import functools

import jax
import jax.numpy as jnp
from jax import lax
from jax.experimental import pallas as pl
from jax.experimental.pallas import tpu as pltpu
from jax.experimental.pallas import tpu_sc as plsc

N = 10000
E = 320000
D = 128
NC, NS = 2, 16
NW = NC * NS
EPW = E // NW
CHUNK = 128
NFULL = EPW // CHUNK
REM = EPW - NFULL * CHUNK
NPAD = 10240
RPT = NPAD // NS
HOPS = RPT // CHUNK


def _sc_aggregate(feat, src, dst, ones_blk, zrow):
    mesh = plsc.VectorSubcoreMesh(core_axis_name="c", subcore_axis_name="s")

    @functools.partial(
        pl.kernel,
        mesh=mesh,
        out_type=[
            jax.ShapeDtypeStruct((NC, NPAD, D), jnp.float32),
            jax.ShapeDtypeStruct((NC, NPAD, D), jnp.float32),
        ],
        scratch_types=[
            pltpu.VMEM((CHUNK,), jnp.int32),
            pltpu.VMEM((CHUNK,), jnp.int32),
            pltpu.VMEM((CHUNK, D), jnp.float32),
            pltpu.VMEM((REM,), jnp.int32),
            pltpu.VMEM((REM,), jnp.int32),
            pltpu.VMEM((REM, D), jnp.float32),
            pltpu.VMEM((CHUNK,), jnp.int32),
            pltpu.VMEM_SHARED((NPAD, D), jnp.float32),
            pltpu.SemaphoreType.DMA,
        ],
    )
    def k(feat_hbm, src_hbm, dst_hbm, ones_hbm, zrow_hbm,
          outp_hbm, outd_hbm,
          sidx, didx, rows, rsidx, rdidx, rrows, zidx, accum, sem):
        c = lax.axis_index("c")
        s = lax.axis_index("s")
        w = s * NC + c
        e_base = w * EPW
        r0 = s * RPT

        def set_zidx(rr):
            for i in range(CHUNK // 16):
                zidx[pl.ds(16 * i, 16)] = rr + 16 * i + lax.iota(jnp.int32, 16)

        def zero_accum():
            pltpu.sync_copy(zrow_hbm, rows)
            for h in range(HOPS):
                set_zidx(r0 + h * CHUNK)
                pltpu.sync_copy(rows, accum.at[zidx])

        def copy_out(out_hbm):
            for h in range(HOPS):
                rr = r0 + h * CHUNK
                set_zidx(rr)
                pltpu.async_copy(accum.at[zidx], rows, sem).wait()
                pltpu.sync_copy(rows, out_hbm.at[c, pl.ds(rr, CHUNK), :])

        zero_accum()
        plsc.subcore_barrier()

        def body(j, carry):
            e0 = e_base + j * CHUNK
            pltpu.sync_copy(src_hbm.at[pl.ds(e0, CHUNK)], sidx)
            pltpu.sync_copy(dst_hbm.at[pl.ds(e0, CHUNK)], didx)
            pltpu.async_copy(feat_hbm.at[sidx], rows, sem).wait()
            pltpu.sync_copy(rows, accum.at[didx], add=True)
            return carry

        lax.fori_loop(0, NFULL, body, 0)

        e0 = e_base + NFULL * CHUNK
        pltpu.sync_copy(src_hbm.at[pl.ds(e0, REM)], rsidx)
        pltpu.sync_copy(dst_hbm.at[pl.ds(e0, REM)], rdidx)
        pltpu.async_copy(feat_hbm.at[rsidx], rrows, sem).wait()
        pltpu.sync_copy(rrows, accum.at[rdidx], add=True)

        plsc.subcore_barrier()
        copy_out(outp_hbm)
        plsc.subcore_barrier()

        zero_accum()
        plsc.subcore_barrier()
        pltpu.sync_copy(ones_hbm, rows)

        def dbody(j, carry):
            e0 = e_base + j * CHUNK
            pltpu.sync_copy(dst_hbm.at[pl.ds(e0, CHUNK)], didx)
            pltpu.sync_copy(rows, accum.at[didx], add=True)
            return carry

        lax.fori_loop(0, NFULL, dbody, 0)
        pltpu.sync_copy(dst_hbm.at[pl.ds(e0, REM)], rdidx)
        pltpu.sync_copy(rows.at[pl.ds(0, REM), :], accum.at[rdidx], add=True)

        plsc.subcore_barrier()
        copy_out(outd_hbm)

    return k(feat, src, dst, ones_blk, zrow)


_ROWS_BLK = 1000


def _tc_body(feat_ref, p_ref, d_ref, ws_ref, wn_ref, b_ref, out_ref):
    neigh = p_ref[0] + p_ref[1]
    deg = d_ref[0, :, 0:1] + d_ref[1, :, 0:1]
    deg = jnp.maximum(deg, 1.0)
    h = jnp.dot(feat_ref[...], ws_ref[...], preferred_element_type=jnp.float32)
    h = h + jnp.dot(neigh / deg, wn_ref[...],
                    preferred_element_type=jnp.float32)
    out_ref[...] = h + b_ref[...]


def _tc_combine(feat, p, d, ws_t, wn_t, bias2d):
    grid = (N // _ROWS_BLK,)
    return pl.pallas_call(
        _tc_body,
        grid=grid,
        in_specs=[
            pl.BlockSpec((_ROWS_BLK, D), lambda i: (i, 0)),
            pl.BlockSpec((NC, _ROWS_BLK, D), lambda i: (0, i, 0)),
            pl.BlockSpec((NC, _ROWS_BLK, D), lambda i: (0, i, 0)),
            pl.BlockSpec((D, D), lambda i: (0, 0)),
            pl.BlockSpec((D, D), lambda i: (0, 0)),
            pl.BlockSpec((1, D), lambda i: (0, 0)),
        ],
        out_specs=pl.BlockSpec((_ROWS_BLK, D), lambda i: (i, 0)),
        out_shape=jax.ShapeDtypeStruct((N, D), jnp.float32),
    )(feat, p, d, ws_t, wn_t, bias2d)


def kernel(feat, edge_index, W_self, W_neigh, bias):
    ones_blk = jnp.ones((CHUNK, D), jnp.float32)
    zrow = jnp.zeros((CHUNK, D), jnp.float32)
    p, d = _sc_aggregate(feat, edge_index[0], edge_index[1], ones_blk, zrow)
    return _tc_combine(feat, p, d, W_self.T, W_neigh.T, bias.reshape(1, D))


# speedup vs baseline: 9.1867x; 1.5760x over previous
import functools

import jax
import jax.numpy as jnp
from jax import lax
from jax.experimental import pallas as pl
from jax.experimental.pallas import tpu as pltpu
from jax.experimental.pallas import tpu_sc as plsc

N = 10000
E = 320000
D = 128
NC, NS = 2, 16
NW = NC * NS
EPW = E // NW
CHUNK = 128
NFULL = EPW // CHUNK
REM = EPW - NFULL * CHUNK
NPAD = 10240
RPT = NPAD // NS
HOPS = RPT // CHUNK


def _sc_aggregate(feat, src, dst, ones_blk, zrow):
    mesh = plsc.VectorSubcoreMesh(core_axis_name="c", subcore_axis_name="s")

    @functools.partial(
        pl.kernel,
        mesh=mesh,
        out_type=[
            jax.ShapeDtypeStruct((NC, NPAD, D), jnp.float32),
            jax.ShapeDtypeStruct((NC, NPAD, D), jnp.float32),
        ],
        scratch_types=[
            pltpu.VMEM((3, CHUNK), jnp.int32),
            pltpu.VMEM((3, CHUNK), jnp.int32),
            pltpu.VMEM((2, CHUNK, D), jnp.float32),
            pltpu.VMEM((REM,), jnp.int32),
            pltpu.VMEM((REM,), jnp.int32),
            pltpu.VMEM((REM, D), jnp.float32),
            pltpu.VMEM((CHUNK,), jnp.int32),
            pltpu.VMEM_SHARED((NPAD, D), jnp.float32),
            pltpu.SemaphoreType.DMA,
            pltpu.SemaphoreType.DMA,
            pltpu.SemaphoreType.DMA,
            pltpu.SemaphoreType.DMA,
            pltpu.SemaphoreType.DMA,
        ],
    )
    def k(feat_hbm, src_hbm, dst_hbm, ones_hbm, zrow_hbm,
          outp_hbm, outd_hbm,
          sidx3, didx3, rows2, rsidx, rdidx, rrows, zidx, accum,
          sem, gsem, isem, ssem0, ssem1):
        ssem = (ssem0, ssem1)
        c = lax.axis_index("c")
        s = lax.axis_index("s")
        w = s * NC + c
        e_base = w * EPW
        r0 = s * RPT

        def set_zidx(rr):
            for i in range(CHUNK // 16):
                zidx[pl.ds(16 * i, 16)] = rr + 16 * i + lax.iota(jnp.int32, 16)

        def zero_accum():
            pltpu.sync_copy(zrow_hbm, rows2.at[0])
            for h in range(HOPS):
                set_zidx(r0 + h * CHUNK)
                pltpu.sync_copy(rows2.at[0], accum.at[zidx])

        def copy_out(out_hbm):
            for h in range(HOPS):
                rr = r0 + h * CHUNK
                set_zidx(rr)
                pltpu.async_copy(accum.at[zidx], rows2.at[0], sem).wait()
                pltpu.sync_copy(rows2.at[0], out_hbm.at[c, pl.ds(rr, CHUNK), :])

        def idx_load(j, buf3, ref_hbm):
            e0 = e_base + j * CHUNK
            return pltpu.async_copy(
                ref_hbm.at[pl.ds(e0, CHUNK)], buf3.at[j % 3], isem)

        zero_accum()
        plsc.subcore_barrier()

        e0 = e_base
        pltpu.sync_copy(src_hbm.at[pl.ds(e0, CHUNK)], sidx3.at[0])
        pltpu.sync_copy(dst_hbm.at[pl.ds(e0, CHUNK)], didx3.at[0])
        g_desc = pltpu.async_copy(feat_hbm.at[sidx3.at[0]], rows2.at[0], gsem)
        i_descs = [idx_load(1, sidx3, src_hbm), idx_load(1, didx3, dst_hbm)]
        s_prev = None
        s_pending = []
        for j in range(NFULL):
            p3, p2 = j % 3, j % 2
            g_desc.wait()
            s_cur = pltpu.async_copy(
                rows2.at[p2], accum.at[didx3.at[p3]], ssem[p2], add=True)
            if j + 1 < NFULL:
                for dsc in i_descs:
                    dsc.wait()
                if s_prev is not None:
                    s_prev.wait()
                q3, q2 = (j + 1) % 3, (j + 1) % 2
                g_desc = pltpu.async_copy(
                    feat_hbm.at[sidx3.at[q3]], rows2.at[q2], gsem)
                if j + 2 < NFULL:
                    i_descs = [idx_load(j + 2, sidx3, src_hbm),
                               idx_load(j + 2, didx3, dst_hbm)]
                else:
                    i_descs = []
                s_prev = s_cur
            else:
                s_pending = [s_cur] + ([s_prev] if s_prev is not None else [])
        for dsc in s_pending:
            dsc.wait()

        e0 = e_base + NFULL * CHUNK
        pltpu.sync_copy(src_hbm.at[pl.ds(e0, REM)], rsidx)
        pltpu.sync_copy(dst_hbm.at[pl.ds(e0, REM)], rdidx)
        pltpu.async_copy(feat_hbm.at[rsidx], rrows, sem).wait()
        pltpu.sync_copy(rrows, accum.at[rdidx], add=True)

        plsc.subcore_barrier()
        copy_out(outp_hbm)
        plsc.subcore_barrier()

        zero_accum()
        plsc.subcore_barrier()
        pltpu.sync_copy(ones_hbm, rows2.at[0])

        pltpu.sync_copy(dst_hbm.at[pl.ds(e_base, CHUNK)], didx3.at[0])
        i_descs = []
        s_prev = None
        s_pending = []
        for j in range(NFULL):
            p3, p2 = j % 3, j % 2
            for dsc in i_descs:
                dsc.wait()
            s_cur = pltpu.async_copy(
                rows2.at[0], accum.at[didx3.at[p3]], ssem[p2], add=True)
            if j + 1 < NFULL:
                i_descs = [idx_load(j + 1, didx3, dst_hbm)]
            else:
                i_descs = []
            if s_prev is not None:
                s_prev.wait()
            if j + 1 < NFULL:
                s_prev = s_cur
            else:
                s_pending = [s_cur]
        for dsc in s_pending:
            dsc.wait()

        pltpu.sync_copy(dst_hbm.at[pl.ds(e_base + NFULL * CHUNK, REM)], rdidx)
        pltpu.sync_copy(rows2.at[0, pl.ds(0, REM), :], accum.at[rdidx],
                        add=True)

        plsc.subcore_barrier()
        copy_out(outd_hbm)

    return k(feat, src, dst, ones_blk, zrow)


_ROWS_BLK = 1000


def _tc_body(feat_ref, p_ref, d_ref, ws_ref, wn_ref, b_ref, out_ref):
    neigh = p_ref[0] + p_ref[1]
    deg = d_ref[0, :, 0:1] + d_ref[1, :, 0:1]
    deg = jnp.maximum(deg, 1.0)
    h = jnp.dot(feat_ref[...], ws_ref[...], preferred_element_type=jnp.float32)
    h = h + jnp.dot(neigh / deg, wn_ref[...],
                    preferred_element_type=jnp.float32)
    out_ref[...] = h + b_ref[...]


def _tc_combine(feat, p, d, ws_t, wn_t, bias2d):
    grid = (N // _ROWS_BLK,)
    return pl.pallas_call(
        _tc_body,
        grid=grid,
        in_specs=[
            pl.BlockSpec((_ROWS_BLK, D), lambda i: (i, 0)),
            pl.BlockSpec((NC, _ROWS_BLK, D), lambda i: (0, i, 0)),
            pl.BlockSpec((NC, _ROWS_BLK, D), lambda i: (0, i, 0)),
            pl.BlockSpec((D, D), lambda i: (0, 0)),
            pl.BlockSpec((D, D), lambda i: (0, 0)),
            pl.BlockSpec((1, D), lambda i: (0, 0)),
        ],
        out_specs=pl.BlockSpec((_ROWS_BLK, D), lambda i: (i, 0)),
        out_shape=jax.ShapeDtypeStruct((N, D), jnp.float32),
    )(feat, p, d, ws_t, wn_t, bias2d)


def kernel(feat, edge_index, W_self, W_neigh, bias):
    ones_blk = jnp.ones((CHUNK, D), jnp.float32)
    zrow = jnp.zeros((CHUNK, D), jnp.float32)
    p, d = _sc_aggregate(feat, edge_index[0], edge_index[1], ones_blk, zrow)
    return _tc_combine(feat, p, d, W_self.T, W_neigh.T, bias.reshape(1, D))
